# baseline (speedup 1.0000x reference)
import functools

import jax
import jax.numpy as jnp
from jax import lax
from jax.experimental import pallas as pl
from jax.experimental.pallas import tpu as pltpu
from jax.experimental.pallas import tpu_sc as plsc

BS, N, V, D = 8, 128, 32000, 256
B, C, DC, DM, RS = 4, 8, 32, 64, 64
G = B * C
T = BS * N
EPS = 1e-6
NW = 32
BPW = T // NW


def _gather_body(table_hbm, idx_hbm, out_hbm, idx_v, rows_v, sem):
    wid = lax.axis_index("s") * 2 + lax.axis_index("c")
    base = wid * BPW
    pltpu.sync_copy(idx_hbm.at[pl.ds(base, BPW)], idx_v)
    pltpu.async_copy(table_hbm.at[idx_v], rows_v, sem).wait()
    pltpu.sync_copy(rows_v, out_hbm.at[pl.ds(base, BPW)])


def _sc_gather(emb, ids_flat):
    mesh = plsc.VectorSubcoreMesh(core_axis_name="c", subcore_axis_name="s")
    f = pl.kernel(
        _gather_body,
        mesh=mesh,
        out_type=jax.ShapeDtypeStruct((T, D), jnp.float32),
        scratch_types=[
            pltpu.VMEM((BPW,), jnp.int32),
            pltpu.VMEM((BPW, D), jnp.float32),
            pltpu.SemaphoreType.DMA,
        ],
    )
    return f(emb, ids_flat)


def _unit_rows(x):
    return x / (jnp.sqrt(jnp.sum(x * x, axis=1, keepdims=True)) + EPS)


def _softmax_rows(x):
    m = jnp.max(x, axis=1, keepdims=True)
    e = jnp.exp(x - m)
    return e / jnp.sum(e, axis=1, keepdims=True)


def _pipeline_body(xg_ref, pos_ref, keep_ref, Wfo_ref, bfo_ref, Wcol_ref,
                   bcol_ref, Wk_ref, Wv_ref, wg_ref, Wrb_ref, pmK_ref,
                   pmV_ref, pma_ref, Wpm_ref, bpm_ref, Wfi_ref, bfi_ref,
                   lng_ref, lnb_ref, lam_ref, out_ref):
    x = xg_ref[...] + pos_ref[...]
    xc = jnp.dot(x, Wfo_ref[...], preferred_element_type=jnp.float32)
    xc = xc + bfo_ref[...]

    Wrb = Wrb_ref[...]
    Wpm = Wpm_ref[...]
    bpm = bpm_ref[...]

    def col_stage(xcols, with_vg):
        hs, ks, vs, gs = [], [], [], []
        for g in range(G):
            xg = xcols[:, g * DC:(g + 1) * DC]
            hg = jax.nn.gelu(
                jnp.dot(xg, Wcol_ref[g], preferred_element_type=jnp.float32)
                + bcol_ref[g][None, :])
            hs.append(hg)
            ks.append(jnp.dot(hg, Wk_ref[g],
                              preferred_element_type=jnp.float32))
            if with_vg:
                vs.append(jnp.dot(hg, Wv_ref[g],
                                  preferred_element_type=jnp.float32))
                gs.append(jax.nn.sigmoid(
                    jnp.sum(hg * wg_ref[g][None, :], axis=1, keepdims=True)))
        return hs, ks, vs, gs

    def attn_round(hs, ks, vs, gs, pmK, pmV, first):
        kns = [_unit_rows(k) for k in ks]
        rb_chunks = {}
        eligK, eligV = {}, {}
        for b in range(B):
            knb = jnp.concatenate([kns[b * C + c] for c in range(C)],
                                  axis=0)
            pkn = _unit_rows(pmK[b])
            scores = lax.dot_general(knb, pkn,
                                     (((1,), (1,)), ((), ())),
                                     preferred_element_type=jnp.float32)
            attn = _softmax_rows(scores)
            read = jnp.dot(attn, pmV[b],
                           preferred_element_type=jnp.float32)
            rb = jnp.dot(read, Wrb, preferred_element_type=jnp.float32)
            for c in range(C):
                rb_chunks[b * C + c] = rb[c * N:(c + 1) * N, :]
            if first:
                kb = jnp.concatenate([ks[b * C + c] for c in range(C)],
                                     axis=0)
                vb = jnp.concatenate([vs[b * C + c] for c in range(C)],
                                     axis=0)
                gb = jnp.concatenate([gs[b * C + c] for c in range(C)],
                                     axis=0)
                gated = gb * attn
                eligK[b] = lax.dot_general(
                    gated, kb, (((0,), (0,)), ((), ())),
                    preferred_element_type=jnp.float32)
                eligV[b] = lax.dot_general(
                    gated, vb, (((0,), (0,)), ((), ())),
                    preferred_element_type=jnp.float32)
        x_out = jnp.concatenate(
            [hs[g] + rb_chunks[g] for g in range(G)], axis=1)
        return x_out, eligK, eligV

    keep = keep_ref[0]
    pmK = [pmK_ref[0, b] * keep for b in range(B)]
    pmV = [pmV_ref[0, b] * keep for b in range(B)]

    hs, ks, vs, gs = col_stage(xc, True)
    x0, eligK, eligV = attn_round(hs, ks, vs, gs, pmK, pmV, True)

    pmK1 = [None] * B
    pmV1 = [None] * B
    for b in range(B):
        eK = eligK[b]
        eV = eligV[b]
        pma_b = pma_ref[0, b][None, :] * keep * 0.99
        es = jnp.mean(jnp.sqrt(jnp.sum(eK * eK, axis=1, keepdims=True)),
                      axis=0, keepdims=True)
        usage = jnp.sum(pma_b, axis=1, keepdims=True)
        content = jnp.mean(eK, axis=0, keepdims=True)
        feats = jnp.concatenate([es, usage, content], axis=1)
        outp = jnp.dot(feats, Wpm,
                       preferred_element_type=jnp.float32) + bpm
        gsb = jax.nn.sigmoid(outp[:, 0:1])
        tau = jax.nn.softplus(outp[:, 1:2]) + 0.5
        ws = _softmax_rows(outp[:, 2:] / tau)
        upd = gsb * jnp.transpose(ws, (1, 0))
        pmK1[b] = pmK[b] + upd * eK
        pmV1[b] = pmV[b] + upd * eV

    hs1, ks1, _, _ = col_stage(x0, False)
    x1, _, _ = attn_round(hs1, ks1, None, None, pmK1, pmV1, False)

    lam = jax.nn.sigmoid(lam_ref[...])
    xcf = (1.0 - lam) * x0 + lam * x1

    xf = jnp.dot(xcf, Wfi_ref[...],
                 preferred_element_type=jnp.float32) + bfi_ref[...]
    mu = jnp.mean(xf, axis=1, keepdims=True)
    var = jnp.mean((xf - mu) ** 2, axis=1, keepdims=True)
    xn = (xf - mu) / jnp.sqrt(var + 1e-5) * lng_ref[...] + lnb_ref[...]
    out_ref[...] = xn


def _pipeline_in_specs():
    full = lambda shape: pl.BlockSpec(shape, lambda s: (0,) * len(shape))
    return [
        pl.BlockSpec((N, D), lambda s: (s, 0)),
        full((N, D)),
        pl.BlockSpec((1, 1, 1), lambda s: (s, 0, 0)),
        full((D, G * DC)),
        full((1, G * DC)),
        full((G, DC, DC)),
        full((G, DC)),
        full((G, DC, DM)),
        full((G, DC, DM)),
        full((G, DC)),
        full((DM, DC)),
        pl.BlockSpec((1, B, RS, DM), lambda s: (s, 0, 0, 0)),
        pl.BlockSpec((1, B, RS, DM), lambda s: (s, 0, 0, 0)),
        pl.BlockSpec((1, B, RS), lambda s: (s, 0, 0)),
        full((DM + 2, 2 + RS)),
        full((1, DM + 2)),
        full((G * DC, D)),
        full((1, D)),
        full((1, D)),
        full((1, D)),
        full((1, 1)),
    ]


VT = 1280


def _logits_body(xn_ref, emb_ref, out_ref):
    out_ref[...] = lax.dot_general(xn_ref[...], emb_ref[...],
                                   (((1,), (1,)), ((), ())),
                                   preferred_element_type=jnp.float32)


def _logits_call(xn, emb):
    return pl.pallas_call(
        _logits_body,
        grid=(V // VT,),
        in_specs=[pl.BlockSpec((T, D), lambda i: (0, 0)),
                  pl.BlockSpec((VT, D), lambda i: (i, 0))],
        out_specs=pl.BlockSpec((T, VT), lambda i: (0, i)),
        out_shape=jax.ShapeDtypeStruct((T, V), jnp.float32),
    )(xn, emb)


def kernel(input_ids, reset_mask, emb, pos, W_fanout, b_fanout, W_col, b_col,
           W_k, W_v, w_gate, W_rb, pm_K0, pm_V0, pm_a0, em_K0, em_V0, em_S0,
           W_pm, b_pm, W_em, b_em, W_fanin, b_fanin, ln_g, ln_b, lambda_logit):
    ids_flat = input_ids.reshape(T).astype(jnp.int32)
    xg = _sc_gather(emb, ids_flat)

    keep = (1.0 - reset_mask.astype(jnp.float32)).reshape(BS, 1, 1)
    xn = pl.pallas_call(
        _pipeline_body,
        grid=(BS,),
        in_specs=_pipeline_in_specs(),
        out_specs=pl.BlockSpec((N, D), lambda s: (s, 0)),
        out_shape=jax.ShapeDtypeStruct((T, D), jnp.float32),
    )(xg, pos, keep, W_fanout, b_fanout.reshape(1, G * DC), W_col, b_col,
      W_k, W_v, w_gate, W_rb, pm_K0, pm_V0, pm_a0, W_pm,
      b_pm.reshape(1, DM + 2), W_fanin, b_fanin.reshape(1, D),
      ln_g.reshape(1, D), ln_b.reshape(1, D), lambda_logit.reshape(1, 1))

    logits = _logits_call(xn, emb).reshape(BS, N, V)
    return (logits, jnp.array(0.0, dtype=jnp.float32))


# speedup vs baseline: 1.0294x; 1.0294x over previous
import jax
import jax.numpy as jnp
from jax import lax
from jax.experimental import pallas as pl
from jax.experimental.pallas import tpu as pltpu
from jax.experimental.pallas import tpu_sc as plsc

BS, N, V, D = 8, 128, 32000, 256
B, C, DC, DM, RS = 4, 8, 32, 64, 64
G = B * C
BD = C * DC
T = BS * N
EPS = 1e-6
NW = 32
BPW = T // NW


def _gather_body(table_hbm, idx_hbm, out_hbm, idx_v, rows_v, sem):
    wid = lax.axis_index("s") * 2 + lax.axis_index("c")
    base = wid * BPW
    pltpu.sync_copy(idx_hbm.at[pl.ds(base, BPW)], idx_v)
    pltpu.async_copy(table_hbm.at[idx_v], rows_v, sem).wait()
    pltpu.sync_copy(rows_v, out_hbm.at[pl.ds(base, BPW)])


def _sc_gather(emb, ids_flat):
    mesh = plsc.VectorSubcoreMesh(core_axis_name="c", subcore_axis_name="s")
    f = pl.kernel(
        _gather_body,
        mesh=mesh,
        out_type=jax.ShapeDtypeStruct((T, D), jnp.float32),
        scratch_types=[
            pltpu.VMEM((BPW,), jnp.int32),
            pltpu.VMEM((BPW, D), jnp.float32),
            pltpu.SemaphoreType.DMA,
        ],
    )
    return f(emb, ids_flat)


def _unit_rows(x):
    return x / (jnp.sqrt(jnp.sum(x * x, axis=1, keepdims=True)) + EPS)


def _dotT(a, b):
    return lax.dot_general(a, b, (((1,), (1,)), ((), ())),
                           preferred_element_type=jnp.float32)


def _dot0(a, b):
    return lax.dot_general(a, b, (((0,), (0,)), ((), ())),
                           preferred_element_type=jnp.float32)


def _mm(a, b):
    return jnp.dot(a, b, preferred_element_type=jnp.float32)


def _pipeline_body(xg_ref, pos_ref, keep_ref, Wfo_ref, bfo_ref, Wcolbd_ref,
                   bcolf_ref, Wkvbd_ref, wgf_ref, Wrb_ref, pmK_ref,
                   pmV_ref, pma_ref, Wpm_ref, bpm_ref, Wfi_ref, bfi_ref,
                   lng_ref, lnb_ref, lam_ref, s256_ref, s512_ref, s128_ref,
                   s2x128_ref, out_ref):
    z64 = jnp.zeros((RS, RS), jnp.float32)
    x = xg_ref[...] + pos_ref[...]
    xc = _mm(x, Wfo_ref[...]) + bfo_ref[...]

    Wrb = Wrb_ref[...]
    Wpm = Wpm_ref[...]
    bpm = bpm_ref[...]
    s256 = s256_ref[...]
    s512 = s512_ref[...]
    s128 = s128_ref[...]
    s2x128 = s2x128_ref[...]
    keep = keep_ref[0]

    def stack_c(arr, base, w):
        return jnp.concatenate(
            [arr[:, base + c * w: base + (c + 1) * w] for c in range(C)],
            axis=0)

    def round_fn(xcols, pmK, pmV, first):
        hs, x_out_bs = [], []
        kbs, vbs, gcols, knbs = [], [], [], []
        for b in range(B):
            xcb = xcols[:, b * BD:(b + 1) * BD]
            hb = jax.nn.gelu(_mm(xcb, Wcolbd_ref[b])
                             + bcolf_ref[:, b * BD:(b + 1) * BD])
            hs.append(hb)
            if first:
                kv = _mm(hb, Wkvbd_ref[b])
                k = kv[:, :C * DM]
            else:
                k = _mm(hb, Wkvbd_ref[b][:, :C * DM])
            n2 = _mm(k * k, s512)
            inv = 1.0 / (jnp.sqrt(n2) + EPS)
            kb = stack_c(kv if first else k, 0, DM)
            invcol = stack_c(inv, 0, 1)
            kbs.append(kb)
            knbs.append(kb * invcol)
            if first:
                vbs.append(stack_c(kv, C * DM, DM))
                gr = _mm(hb * wgf_ref[:, b * BD:(b + 1) * BD], s256)
                gcols.append(stack_c(jax.nn.sigmoid(gr), 0, 1))

        eligK, eligV = {}, {}
        for p in range(B // 2):
            b0, b1 = 2 * p, 2 * p + 1
            kn_pair = jnp.concatenate([knbs[b0], knbs[b1]], axis=1)
            pkn0 = _unit_rows(pmK[b0])
            pkn1 = _unit_rows(pmK[b1])
            PK = jnp.concatenate(
                [jnp.concatenate([pkn0, z64], axis=1),
                 jnp.concatenate([z64, pkn1], axis=1)], axis=0)
            scores = _dotT(kn_pair, PK)
            e = jnp.exp(scores)
            inv2 = 1.0 / _mm(e, s128)
            attn_pair = e * _mm(inv2, s2x128)
            for half, b in ((0, b0), (1, b1)):
                attn = attn_pair[:, half * RS:(half + 1) * RS]
                pvrb = _mm(pmV[b], Wrb)
                rb = _mm(attn, pvrb)
                xob = jnp.concatenate(
                    [rb[c * N:(c + 1) * N, :] for c in range(C)], axis=1)
                x_out_bs.append((b, hs[b] + xob))
                if first:
                    kvb = jnp.concatenate([kbs[b], vbs[b]], axis=1)
                    gated = gcols[b] * attn
                    eKV = _dot0(gated, kvb)
                    eligK[b] = eKV[:, :DM]
                    eligV[b] = eKV[:, DM:]
        x_out = jnp.concatenate(
            [xb for _, xb in sorted(x_out_bs)], axis=1)
        return x_out, eligK, eligV

    pmK = [pmK_ref[0, b] * keep for b in range(B)]
    pmV = [pmV_ref[0, b] * keep for b in range(B)]
    x0, eligK, eligV = round_fn(xc, pmK, pmV, True)

    pmK1, pmV1 = [None] * B, [None] * B
    for b in range(B):
        eK = eligK[b]
        eV = eligV[b]
        pma_b = pma_ref[0, b][None, :] * keep * 0.99
        es = jnp.mean(jnp.sqrt(jnp.sum(eK * eK, axis=1, keepdims=True)),
                      axis=0, keepdims=True)
        usage = jnp.sum(pma_b, axis=1, keepdims=True)
        content = jnp.mean(eK, axis=0, keepdims=True)
        feats = jnp.concatenate([es, usage, content], axis=1)
        outp = _mm(feats, Wpm) + bpm
        gsb = jax.nn.sigmoid(outp[:, 0:1])
        tau = jax.nn.softplus(outp[:, 1:2]) + 0.5
        sl = outp[:, 2:] / tau
        esl = jnp.exp(sl - jnp.max(sl, axis=1, keepdims=True))
        ws = esl / jnp.sum(esl, axis=1, keepdims=True)
        upd = gsb * jnp.transpose(ws, (1, 0))
        pmK1[b] = pmK[b] + upd * eK
        pmV1[b] = pmV[b] + upd * eV

    x1, _, _ = round_fn(x0, pmK1, pmV1, False)

    lam = jax.nn.sigmoid(lam_ref[...])
    xcf = (1.0 - lam) * x0 + lam * x1

    xf = _mm(xcf, Wfi_ref[...]) + bfi_ref[...]
    mu = jnp.mean(xf, axis=1, keepdims=True)
    var = jnp.mean((xf - mu) ** 2, axis=1, keepdims=True)
    xn = (xf - mu) / jnp.sqrt(var + 1e-5) * lng_ref[...] + lnb_ref[...]
    out_ref[...] = xn


def _pipeline_in_specs():
    full = lambda shape: pl.BlockSpec(shape, lambda s: (0,) * len(shape))
    return [
        pl.BlockSpec((N, D), lambda s: (s, 0)),
        full((N, D)),
        pl.BlockSpec((1, 1, 1), lambda s: (s, 0, 0)),
        full((D, G * DC)),
        full((1, G * DC)),
        full((B, BD, BD)),
        full((1, G * DC)),
        full((B, BD, 2 * C * DM)),
        full((1, G * DC)),
        full((DM, DC)),
        pl.BlockSpec((1, B, RS, DM), lambda s: (s, 0, 0, 0)),
        pl.BlockSpec((1, B, RS, DM), lambda s: (s, 0, 0, 0)),
        pl.BlockSpec((1, B, RS), lambda s: (s, 0, 0)),
        full((DM + 2, 2 + RS)),
        full((1, DM + 2)),
        full((G * DC, D)),
        full((1, D)),
        full((1, D)),
        full((1, D)),
        full((1, 1)),
        full((BD, C)),
        full((C * DM, C)),
        full((2 * RS, 2)),
        full((2, 2 * RS)),
    ]


VT = 3200


def _logits_body(xn_ref, emb_ref, out_ref):
    out_ref[...] = _dotT(xn_ref[...], emb_ref[...])


def _logits_call(xn, emb):
    return pl.pallas_call(
        _logits_body,
        grid=(V // VT,),
        in_specs=[pl.BlockSpec((T, D), lambda i: (0, 0)),
                  pl.BlockSpec((VT, D), lambda i: (i, 0))],
        out_specs=pl.BlockSpec((T, VT), lambda i: (0, i)),
        out_shape=jax.ShapeDtypeStruct((T, V), jnp.float32),
    )(xn, emb)


import functools


def kernel(input_ids, reset_mask, emb, pos, W_fanout, b_fanout, W_col, b_col,
           W_k, W_v, w_gate, W_rb, pm_K0, pm_V0, pm_a0, em_K0, em_V0, em_S0,
           W_pm, b_pm, W_em, b_em, W_fanin, b_fanin, ln_g, ln_b, lambda_logit):
    ids_flat = input_ids.reshape(T).astype(jnp.int32)
    xg = _sc_gather(emb, ids_flat)

    keep = (1.0 - reset_mask.astype(jnp.float32)).reshape(BS, 1, 1)

    eye_c = jnp.eye(C, dtype=jnp.float32)
    Wcol_bd = jnp.einsum('bcij,cd->bcidj', W_col.reshape(B, C, DC, DC),
                         eye_c).reshape(B, BD, BD)
    Wk_bd = jnp.einsum('bcim,cd->bcidm', W_k.reshape(B, C, DC, DM),
                       eye_c).reshape(B, BD, C * DM)
    Wv_bd = jnp.einsum('bcim,cd->bcidm', W_v.reshape(B, C, DC, DM),
                       eye_c).reshape(B, BD, C * DM)
    Wkv_bd = jnp.concatenate([Wk_bd, Wv_bd], axis=2)

    s256 = (jnp.arange(BD)[:, None] // DC ==
            jnp.arange(C)[None, :]).astype(jnp.float32)
    s512 = (jnp.arange(C * DM)[:, None] // DM ==
            jnp.arange(C)[None, :]).astype(jnp.float32)
    s128 = (jnp.arange(2 * RS)[:, None] // RS ==
            jnp.arange(2)[None, :]).astype(jnp.float32)
    s2x128 = jnp.transpose(s128)

    xn = pl.pallas_call(
        _pipeline_body,
        grid=(BS,),
        in_specs=_pipeline_in_specs(),
        out_specs=pl.BlockSpec((N, D), lambda s: (s, 0)),
        out_shape=jax.ShapeDtypeStruct((T, D), jnp.float32),
    )(xg, pos, keep, W_fanout, b_fanout.reshape(1, G * DC), Wcol_bd,
      b_col.reshape(1, G * DC), Wkv_bd, w_gate.reshape(1, G * DC), W_rb,
      pm_K0, pm_V0, pm_a0, W_pm, b_pm.reshape(1, DM + 2), W_fanin,
      b_fanin.reshape(1, D), ln_g.reshape(1, D), ln_b.reshape(1, D),
      lambda_logit.reshape(1, 1), s256, s512, s128, s2x128)

    logits = _logits_call(xn, emb).reshape(BS, N, V)
    return (logits, jnp.array(0.0, dtype=jnp.float32))


# speedup vs baseline: 1.2255x; 1.1905x over previous
import jax
import jax.numpy as jnp
from jax import lax
from jax.experimental import pallas as pl
from jax.experimental.pallas import tpu as pltpu
from jax.experimental.pallas import tpu_sc as plsc

BS, N, V, D = 8, 128, 32000, 256
B, C, DC, DM, RS = 4, 8, 32, 64, 64
G = B * C
BD = C * DC
KW = C * DM
T = BS * N
SPS = 4
SN = SPS * N
EPS = 1e-6
NW = 32
BPW = T // NW


def _gather_body(table_hbm, idx_hbm, out_hbm, idx_v, rows_v, sem):
    wid = lax.axis_index("s") * 2 + lax.axis_index("c")
    base = wid * BPW
    pltpu.sync_copy(idx_hbm.at[pl.ds(base, BPW)], idx_v)
    pltpu.async_copy(table_hbm.at[idx_v], rows_v, sem).wait()
    pltpu.sync_copy(rows_v, out_hbm.at[pl.ds(base, BPW)])


def _sc_gather(emb, ids_flat):
    mesh = plsc.VectorSubcoreMesh(core_axis_name="c", subcore_axis_name="s")
    f = pl.kernel(
        _gather_body,
        mesh=mesh,
        out_type=jax.ShapeDtypeStruct((T, D), jnp.float32),
        scratch_types=[
            pltpu.VMEM((BPW,), jnp.int32),
            pltpu.VMEM((BPW, D), jnp.float32),
            pltpu.SemaphoreType.DMA,
        ],
    )
    return f(emb, ids_flat)


def _unit_rows(x):
    return x / (jnp.sqrt(jnp.sum(x * x, axis=1, keepdims=True)) + EPS)


def _dotT(a, b):
    return lax.dot_general(a, b, (((1,), (1,)), ((), ())),
                           preferred_element_type=jnp.float32)


def _dot0(a, b):
    return lax.dot_general(a, b, (((0,), (0,)), ((), ())),
                           preferred_element_type=jnp.float32)


def _mm(a, b):
    return jnp.dot(a, b, preferred_element_type=jnp.float32)


def _pipeline_body(xg_ref, pos_ref, keep_ref, Mfc_ref, bfc_ref, Wcolbd_ref,
                   bcolf_ref, Wkvbd_ref, wgf_ref, Wrb_ref, pmK_ref,
                   pmV_ref, pma_ref, Wpm_ref, bpm_ref, Wfi_ref, bfi_ref,
                   lng_ref, lnb_ref, lam_ref, s512_ref, s512T_ref, s256_ref,
                   s2564_ref, out_ref):
    x = xg_ref[...] + pos_ref[...]
    Wrb = Wrb_ref[...]
    s512 = s512_ref[...]
    s512T = s512T_ref[...]
    keeps = [keep_ref[si] for si in range(SPS)]

    def round_fn(xin, pmKs, pmVs, first):
        x_out_bs = []
        eligKV = [[] for _ in range(SPS)]
        for b in range(B):
            if first:
                hb = jax.nn.gelu(_mm(xin, Mfc_ref[b])
                                 + bfc_ref[:, b * BD:(b + 1) * BD])
                kv = _mm(hb, Wkvbd_ref[b])
                k = kv[:, :KW]
            else:
                hb = jax.nn.gelu(_mm(xin[:, b * BD:(b + 1) * BD],
                                     Wcolbd_ref[b])
                                 + bcolf_ref[:, b * BD:(b + 1) * BD])
                k = _mm(hb, Wkvbd_ref[b][:, :KW])
            n2 = _mm(k * k, s512)
            inv = 1.0 / (jnp.sqrt(n2) + EPS)
            kn = k * _mm(inv, s512T)
            if first:
                gr = jax.nn.sigmoid(
                    _mm(hb * wgf_ref[:, b * BD:(b + 1) * BD], s256_ref[...]))
                gate_exp = _mm(gr, s512T)
            xobs = []
            for si in range(SPS):
                r0, r1 = si * N, (si + 1) * N
                kns = kn[r0:r1]
                pkn = _unit_rows(pmKs[si][b])
                pvrb = _mm(pmVs[si][b], Wrb)
                scores = jnp.concatenate(
                    [_dotT(kns[:, c * DM:(c + 1) * DM], pkn)
                     for c in range(C)], axis=1)
                e = jnp.exp(scores)
                inv2 = 1.0 / _mm(e, s512)
                attn = e * _mm(inv2, s512T)
                xobs.append(jnp.concatenate(
                    [_mm(attn[:, c * RS:(c + 1) * RS], pvrb)
                     for c in range(C)], axis=1))
                if first:
                    gated = attn * gate_exp[r0:r1]
                    gatedT = jnp.transpose(gated, (1, 0))
                    kvs = kv[r0:r1]
                    acc = None
                    for c in range(C):
                        kvc = jnp.concatenate(
                            [kvs[:, c * DM:(c + 1) * DM],
                             kvs[:, KW + c * DM:KW + (c + 1) * DM]], axis=1)
                        d = _mm(gatedT[c * DM:(c + 1) * DM, :], kvc)
                        acc = d if acc is None else acc + d
                    eligKV[si].append(acc)
            x_out_bs.append(hb + jnp.concatenate(xobs, axis=0))
        return jnp.concatenate(x_out_bs, axis=1), eligKV

    pmKs = [[pmK_ref[si, b] * keeps[si] for b in range(B)]
            for si in range(SPS)]
    pmVs = [[pmV_ref[si, b] * keeps[si] for b in range(B)]
            for si in range(SPS)]
    x0, eligKV = round_fn(x, pmKs, pmVs, True)

    s2564 = s2564_ref[...]
    Wpm = Wpm_ref[...]
    bpm = bpm_ref[...]
    pmK1s, pmV1s = [], []
    for si in range(SPS):
        eKV = jnp.concatenate(eligKV[si], axis=0)
        eK = eKV[:, :DM]
        nrm = jnp.sqrt(jnp.sum(eK * eK, axis=1, keepdims=True))
        es = _dot0(s2564, nrm) * (1.0 / RS)
        content = _dot0(s2564, eK) * (1.0 / RS)
        pma = pma_ref[si] * (0.99 * keeps[si])
        usage = jnp.sum(pma, axis=1, keepdims=True)
        feats = jnp.concatenate([es, usage, content], axis=1)
        outp = _mm(feats, Wpm) + bpm
        gs = jax.nn.sigmoid(outp[:, 0:1])
        tau = jax.nn.softplus(outp[:, 1:2]) + 0.5
        sl = outp[:, 2:] / tau
        esl = jnp.exp(sl - jnp.max(sl, axis=1, keepdims=True))
        ws = esl / jnp.sum(esl, axis=1, keepdims=True)
        updT = jnp.transpose(gs * ws, (1, 0))
        pmK1s.append([pmKs[si][b] + updT[:, b:b + 1] * eligKV[si][b][:, :DM]
                      for b in range(B)])
        pmV1s.append([pmVs[si][b] + updT[:, b:b + 1] * eligKV[si][b][:, DM:]
                      for b in range(B)])

    x1, _ = round_fn(x0, pmK1s, pmV1s, False)

    lam = jax.nn.sigmoid(lam_ref[...])
    xcf = (1.0 - lam) * x0 + lam * x1

    xf = _mm(xcf, Wfi_ref[...]) + bfi_ref[...]
    mu = jnp.mean(xf, axis=1, keepdims=True)
    var = jnp.mean((xf - mu) ** 2, axis=1, keepdims=True)
    xn = (xf - mu) / jnp.sqrt(var + 1e-5) * lng_ref[...] + lnb_ref[...]
    out_ref[...] = xn


def _pipeline_in_specs():
    full = lambda shape: pl.BlockSpec(shape, lambda p: (0,) * len(shape))
    return [
        pl.BlockSpec((SN, D), lambda p: (p, 0)),
        full((SN, D)),
        pl.BlockSpec((SPS, 1, 1), lambda p: (p, 0, 0)),
        full((B, D, BD)),
        full((1, G * DC)),
        full((B, BD, BD)),
        full((1, G * DC)),
        full((B, BD, 2 * KW)),
        full((1, G * DC)),
        full((DM, DC)),
        pl.BlockSpec((SPS, B, RS, DM), lambda p: (p, 0, 0, 0)),
        pl.BlockSpec((SPS, B, RS, DM), lambda p: (p, 0, 0, 0)),
        pl.BlockSpec((SPS, B, RS), lambda p: (p, 0, 0)),
        full((DM + 2, 2 + RS)),
        full((1, DM + 2)),
        full((G * DC, D)),
        full((1, D)),
        full((1, D)),
        full((1, D)),
        full((1, 1)),
        full((KW, C)),
        full((C, KW)),
        full((BD, C)),
        full((BD, B)),
    ]


VT = 3200


def _logits_body(xn_ref, emb_ref, out_ref):
    out_ref[...] = _dotT(xn_ref[...], emb_ref[...])


def _logits_call(xn, emb):
    return pl.pallas_call(
        _logits_body,
        grid=(V // VT,),
        in_specs=[pl.BlockSpec((T, D), lambda i: (0, 0)),
                  pl.BlockSpec((VT, D), lambda i: (i, 0))],
        out_specs=pl.BlockSpec((T, VT), lambda i: (0, i)),
        out_shape=jax.ShapeDtypeStruct((T, V), jnp.float32),
    )(xn, emb)


def kernel(input_ids, reset_mask, emb, pos, W_fanout, b_fanout, W_col, b_col,
           W_k, W_v, w_gate, W_rb, pm_K0, pm_V0, pm_a0, em_K0, em_V0, em_S0,
           W_pm, b_pm, W_em, b_em, W_fanin, b_fanin, ln_g, ln_b, lambda_logit):
    ids_flat = input_ids.reshape(T).astype(jnp.int32)
    xg = _sc_gather(emb, ids_flat)

    keep = (1.0 - reset_mask.astype(jnp.float32)).reshape(BS, 1, 1)
    pos4 = jnp.tile(pos, (SPS, 1))

    eye_c = jnp.eye(C, dtype=jnp.float32)
    Wcol_bd = jnp.einsum('bcij,cd->bcidj', W_col.reshape(B, C, DC, DC),
                         eye_c).reshape(B, BD, BD)
    Wk_bd = jnp.einsum('bcim,cd->bcidm', W_k.reshape(B, C, DC, DM),
                       eye_c).reshape(B, BD, KW)
    Wv_bd = jnp.einsum('bcim,cd->bcidm', W_v.reshape(B, C, DC, DM),
                       eye_c).reshape(B, BD, KW)
    Wkv_bd = jnp.concatenate([Wk_bd, Wv_bd], axis=2)
    Mfc = jnp.einsum('bdi,bij->bdj',
                     jnp.transpose(W_fanout.reshape(D, B, BD), (1, 0, 2)),
                     Wcol_bd)
    bfc = (jnp.einsum('bi,bij->bj',
                      b_fanout.reshape(B, BD), Wcol_bd).reshape(1, G * DC)
           + b_col.reshape(1, G * DC))

    s512 = (jnp.arange(KW)[:, None] // DM ==
            jnp.arange(C)[None, :]).astype(jnp.float32)
    s512T = jnp.transpose(s512)
    s256 = (jnp.arange(BD)[:, None] // DC ==
            jnp.arange(C)[None, :]).astype(jnp.float32)
    s2564 = (jnp.arange(BD)[:, None] // RS ==
             jnp.arange(B)[None, :]).astype(jnp.float32)

    xn = pl.pallas_call(
        _pipeline_body,
        grid=(BS // SPS,),
        in_specs=_pipeline_in_specs(),
        out_specs=pl.BlockSpec((SN, D), lambda p: (p, 0)),
        out_shape=jax.ShapeDtypeStruct((T, D), jnp.float32),
    )(xg, pos4, keep, Mfc, bfc, Wcol_bd, b_col.reshape(1, G * DC), Wkv_bd,
      w_gate.reshape(1, G * DC), W_rb, pm_K0, pm_V0, pm_a0, W_pm,
      b_pm.reshape(1, DM + 2), W_fanin, b_fanin.reshape(1, D),
      ln_g.reshape(1, D), ln_b.reshape(1, D), lambda_logit.reshape(1, 1),
      s512, s512T, s256, s2564)

    logits = _logits_call(xn, emb).reshape(BS, N, V)
    return (logits, jnp.array(0.0, dtype=jnp.float32))
